# baseline (device time: 48746 ns/iter reference)
import jax
import jax.numpy as jnp
from jax import lax
from jax.experimental import pallas as pl
from jax.experimental.pallas import tpu as pltpu

N_DEV = 4
HEADS = 8
DH = 128
SCALE = 0.08838834764831843

AG_R1, AG_L1, AG_R2, AG_L2, RS_R1, RS_L1, RS_R2, RS_L2 = range(8)


def kernel(x, Wq, Wo, Wk, Wv):
    _, s_per, d = x.shape
    seq = N_DEV * s_per
    s_half = s_per // 2

    def body(x_ref, wq_ref, wo_ref, wk_ref, wv_ref, out_ref,
             xg_ref, qg_ref, kg_ref, vg_ref,
             sbD_ref, rb1_ref, rb2_ref, mR_ref, mL_ref, fL_ref, fR_ref,
             snd, rcv):
        p = lax.axis_index("i")
        left = lax.rem(p + N_DEV - 1, N_DEV)
        right = lax.rem(p + 1, N_DEV)
        cm1, cp1, cp2 = left, right, lax.rem(p + 2, N_DEV)

        barrier = pltpu.get_barrier_semaphore()
        for nbr in (left, right):
            pl.semaphore_signal(barrier, inc=1, device_id=(nbr,),
                                device_id_type=pl.DeviceIdType.MESH)
        pl.semaphore_wait(barrier, 2)

        def rdma(src, dst, i, dev):
            return pltpu.make_async_remote_copy(
                src_ref=src, dst_ref=dst,
                send_sem=snd.at[i], recv_sem=rcv.at[i],
                device_id=(dev,), device_id_type=pl.DeviceIdType.MESH)

        bf16 = jnp.bfloat16
        wq_bf = wq_ref[...].astype(bf16)
        wk_bf = wk_ref[...].astype(bf16)
        wv_bf = wv_ref[...].astype(bf16)
        wo_bf = wo_ref[...].astype(bf16)

        def project(c):
            rows = pl.ds(c * s_per, s_per)
            xc = xg_ref[rows, :]
            qg_ref[rows, :] = (jnp.dot(
                xc, wq_bf, preferred_element_type=jnp.float32)
                * SCALE).astype(bf16)
            kg_ref[rows, :] = jnp.dot(
                xc, wk_bf, preferred_element_type=jnp.float32).astype(bf16)
            vg_ref[rows, :] = jnp.dot(
                xc, wv_bf, preferred_element_type=jnp.float32).astype(bf16)

        own = pl.ds(p * s_per, s_per)
        xg_ref[own, :] = x_ref[0].astype(bf16)
        aR1 = rdma(xg_ref.at[own, :], xg_ref.at[own, :], AG_R1, right)
        aL1 = rdma(xg_ref.at[own, :], xg_ref.at[own, :], AG_L1, left)
        aR1.start()
        aL1.start()
        project(p)
        aR1.wait_recv()
        aL1.wait_recv()
        h1m1 = pl.ds(cm1 * s_per, s_half)
        h2p1 = pl.ds(cp1 * s_per + s_half, s_half)
        aR2 = rdma(xg_ref.at[h1m1, :], xg_ref.at[h1m1, :], AG_R2, right)
        aL2 = rdma(xg_ref.at[h2p1, :], xg_ref.at[h2p1, :], AG_L2, left)
        aR2.start()
        aL2.start()
        project(cm1)
        project(cp1)
        aR2.wait_recv()
        aL2.wait_recv()
        project(cp2)

        ones_v = jnp.ones((seq,), dtype=bf16)

        def attn_part(c):
            rows = pl.ds(c * s_per, s_per)
            ohs = []
            for h in range(HEADS):
                cols = slice(h * DH, (h + 1) * DH)
                qh = qg_ref[rows, cols]
                kh = kg_ref[:, cols]
                vh = vg_ref[:, cols]
                s = lax.dot_general(qh, kh, (((1,), (1,)), ((), ())),
                                    preferred_element_type=jnp.float32)
                e = jnp.exp(s).astype(bf16)
                l = jnp.dot(e, ones_v, preferred_element_type=jnp.float32)
                oh = (jnp.dot(e, vh, preferred_element_type=jnp.float32)
                      * (1.0 / l)[:, None])
                ohs.append(oh.astype(bf16))
            attn_c = jnp.concatenate(ohs, axis=1)
            return jnp.dot(attn_c, wo_bf, preferred_element_type=jnp.float32)

        sbD_ref[...] = attn_part(cp2).astype(bf16)
        bR1 = rdma(sbD_ref.at[pl.ds(0, s_half), :], rb1_ref, RS_R1, right)
        bL1 = rdma(sbD_ref.at[pl.ds(s_half, s_half), :], rb2_ref, RS_L1, left)
        bR1.start()
        bL1.start()

        partR = attn_part(cp1)
        bR1.wait_recv()
        mR_ref[pl.ds(0, s_half), :] = (
            partR[:s_half] + rb1_ref[...]).astype(bf16)
        mR_ref[pl.ds(s_half, s_half), :] = partR[s_half:].astype(bf16)
        bR2 = rdma(mR_ref, fL_ref, RS_R2, right)
        bR2.start()

        partL = attn_part(cm1)
        bL1.wait_recv()
        mL_ref[pl.ds(s_half, s_half), :] = (
            partL[s_half:] + rb2_ref[...]).astype(bf16)
        mL_ref[pl.ds(0, s_half), :] = partL[:s_half].astype(bf16)
        bL2 = rdma(mL_ref, fR_ref, RS_L2, left)
        bL2.start()

        part0 = attn_part(p)
        bR2.wait_recv()
        bL2.wait_recv()
        out_ref[0] = part0 + fL_ref[...] + fR_ref[...]

        for desc in (aR1, aL1, aR2, aL2, bR1, bL1, bR2, bL2):
            desc.wait_send()

    return pl.pallas_call(
        body,
        out_shape=jax.ShapeDtypeStruct((1, s_per, d), jnp.float32),
        in_specs=[pl.BlockSpec(memory_space=pltpu.VMEM)] * 5,
        out_specs=pl.BlockSpec(memory_space=pltpu.VMEM),
        scratch_shapes=[
            pltpu.VMEM((seq, d), jnp.bfloat16),
            pltpu.VMEM((seq, d), jnp.bfloat16),
            pltpu.VMEM((seq, d), jnp.bfloat16),
            pltpu.VMEM((seq, d), jnp.bfloat16),
            pltpu.VMEM((s_per, d), jnp.bfloat16),
            pltpu.VMEM((s_half, d), jnp.bfloat16),
            pltpu.VMEM((s_half, d), jnp.bfloat16),
            pltpu.VMEM((s_per, d), jnp.bfloat16),
            pltpu.VMEM((s_per, d), jnp.bfloat16),
            pltpu.VMEM((s_per, d), jnp.bfloat16),
            pltpu.VMEM((s_per, d), jnp.bfloat16),
            pltpu.SemaphoreType.DMA((8,)),
            pltpu.SemaphoreType.DMA((8,)),
        ],
        compiler_params=pltpu.CompilerParams(
            collective_id=0, vmem_limit_bytes=100 * 1024 * 1024),
    )(x, Wq, Wo, Wk, Wv)


# device time: 47546 ns/iter; 1.0252x vs baseline; 1.0252x over previous
import jax
import jax.numpy as jnp
from jax import lax
from jax.experimental import pallas as pl
from jax.experimental.pallas import tpu as pltpu

N_DEV = 4
HEADS = 8
DH = 128
SCALE = 0.08838834764831843

AG_R1, AG_L1, AG_R2, AG_L2, RS_R1, RS_L1, RS_R2, RS_L2 = range(8)


def kernel(x, Wq, Wo, Wk, Wv):
    _, s_per, d = x.shape
    seq = N_DEV * s_per
    s_half = s_per // 2

    def body(x_ref, wq_ref, wo_ref, wk_ref, wv_ref, out_ref,
             xg_ref, qg_ref, kg_ref, vg_ref,
             sbD_ref, rb1_ref, rb2_ref, mR_ref, mL_ref, fL_ref, fR_ref,
             snd, rcv):
        p = lax.axis_index("i")
        left = lax.rem(p + N_DEV - 1, N_DEV)
        right = lax.rem(p + 1, N_DEV)
        cm1, cp1, cp2 = left, right, lax.rem(p + 2, N_DEV)

        barrier = pltpu.get_barrier_semaphore()
        for nbr in (left, right):
            pl.semaphore_signal(barrier, inc=1, device_id=(nbr,),
                                device_id_type=pl.DeviceIdType.MESH)
        pl.semaphore_wait(barrier, 2)

        def rdma(src, dst, i, dev):
            return pltpu.make_async_remote_copy(
                src_ref=src, dst_ref=dst,
                send_sem=snd.at[i], recv_sem=rcv.at[i],
                device_id=(dev,), device_id_type=pl.DeviceIdType.MESH)

        bf16 = jnp.bfloat16

        own = pl.ds(p * s_per, s_per)
        xg_ref[own, :] = x_ref[0].astype(bf16)
        aR1 = rdma(xg_ref.at[own, :], xg_ref.at[own, :], AG_R1, right)
        aL1 = rdma(xg_ref.at[own, :], xg_ref.at[own, :], AG_L1, left)
        aR1.start()
        aL1.start()

        wqkv_bf = jnp.concatenate(
            [(wq_ref[...] * SCALE).astype(bf16),
             wk_ref[...].astype(bf16),
             wv_ref[...].astype(bf16)], axis=1)
        wo_bf = wo_ref[...].astype(bf16)

        def project(c):
            rows = pl.ds(c * s_per, s_per)
            qkv = jnp.dot(xg_ref[rows, :], wqkv_bf,
                          preferred_element_type=jnp.float32)
            qg_ref[rows, :] = qkv[:, :d].astype(bf16)
            kg_ref[rows, :] = qkv[:, d:2 * d].astype(bf16)
            vg_ref[rows, :] = qkv[:, 2 * d:].astype(bf16)

        project(p)
        aR1.wait_recv()
        aL1.wait_recv()
        h1m1 = pl.ds(cm1 * s_per, s_half)
        h2p1 = pl.ds(cp1 * s_per + s_half, s_half)
        aR2 = rdma(xg_ref.at[h1m1, :], xg_ref.at[h1m1, :], AG_R2, right)
        aL2 = rdma(xg_ref.at[h2p1, :], xg_ref.at[h2p1, :], AG_L2, left)
        aR2.start()
        aL2.start()
        project(cm1)
        project(cp1)
        aR2.wait_recv()
        aL2.wait_recv()
        project(cp2)

        ones_v = jnp.ones((seq,), dtype=bf16)

        def attn_part(c):
            rows = pl.ds(c * s_per, s_per)
            ohs = []
            for h in range(HEADS):
                cols = slice(h * DH, (h + 1) * DH)
                qh = qg_ref[rows, cols]
                kh = kg_ref[:, cols]
                vh = vg_ref[:, cols]
                s = lax.dot_general(qh, kh, (((1,), (1,)), ((), ())),
                                    preferred_element_type=jnp.float32)
                e = jnp.exp(s).astype(bf16)
                l = jnp.dot(e, ones_v, preferred_element_type=jnp.float32)
                oh = (jnp.dot(e, vh, preferred_element_type=jnp.float32)
                      * (1.0 / l)[:, None])
                ohs.append(oh.astype(bf16))
            attn_c = jnp.concatenate(ohs, axis=1)
            return jnp.dot(attn_c, wo_bf, preferred_element_type=jnp.float32)

        sbD_ref[...] = attn_part(cp2).astype(bf16)
        bR1 = rdma(sbD_ref.at[pl.ds(0, s_half), :], rb1_ref, RS_R1, right)
        bL1 = rdma(sbD_ref.at[pl.ds(s_half, s_half), :], rb2_ref, RS_L1, left)
        bR1.start()
        bL1.start()

        partR = attn_part(cp1)
        bR1.wait_recv()
        mR_ref[pl.ds(0, s_half), :] = (
            partR[:s_half] + rb1_ref[...]).astype(bf16)
        mR_ref[pl.ds(s_half, s_half), :] = partR[s_half:].astype(bf16)
        bR2 = rdma(mR_ref, fL_ref, RS_R2, right)
        bR2.start()

        partL = attn_part(cm1)
        bL1.wait_recv()
        mL_ref[pl.ds(s_half, s_half), :] = (
            partL[s_half:] + rb2_ref[...]).astype(bf16)
        mL_ref[pl.ds(0, s_half), :] = partL[:s_half].astype(bf16)
        bL2 = rdma(mL_ref, fR_ref, RS_L2, left)
        bL2.start()

        part0 = attn_part(p)
        bR2.wait_recv()
        bL2.wait_recv()
        out_ref[0] = part0 + fL_ref[...] + fR_ref[...]

        for desc in (aR1, aL1, aR2, aL2, bR1, bL1, bR2, bL2):
            desc.wait_send()

    return pl.pallas_call(
        body,
        out_shape=jax.ShapeDtypeStruct((1, s_per, d), jnp.float32),
        in_specs=[pl.BlockSpec(memory_space=pltpu.VMEM)] * 5,
        out_specs=pl.BlockSpec(memory_space=pltpu.VMEM),
        scratch_shapes=[
            pltpu.VMEM((seq, d), jnp.bfloat16),
            pltpu.VMEM((seq, d), jnp.bfloat16),
            pltpu.VMEM((seq, d), jnp.bfloat16),
            pltpu.VMEM((seq, d), jnp.bfloat16),
            pltpu.VMEM((s_per, d), jnp.bfloat16),
            pltpu.VMEM((s_half, d), jnp.bfloat16),
            pltpu.VMEM((s_half, d), jnp.bfloat16),
            pltpu.VMEM((s_per, d), jnp.bfloat16),
            pltpu.VMEM((s_per, d), jnp.bfloat16),
            pltpu.VMEM((s_per, d), jnp.bfloat16),
            pltpu.VMEM((s_per, d), jnp.bfloat16),
            pltpu.SemaphoreType.DMA((8,)),
            pltpu.SemaphoreType.DMA((8,)),
        ],
        compiler_params=pltpu.CompilerParams(
            collective_id=0, vmem_limit_bytes=100 * 1024 * 1024),
    )(x, Wq, Wo, Wk, Wv)


# device time: 45426 ns/iter; 1.0731x vs baseline; 1.0467x over previous
import jax
import jax.numpy as jnp
from jax import lax
from jax.experimental import pallas as pl
from jax.experimental.pallas import tpu as pltpu

N_DEV = 4
HEADS = 8
DH = 128
SCALE = 0.08838834764831843

AG_R1, AG_L1, AG_R2, AG_L2, RS_R1, RS_L1, RS_R2, RS_L2 = range(8)


def kernel(x, Wq, Wo, Wk, Wv):
    _, s_per, d = x.shape
    seq = N_DEV * s_per
    s_half = s_per // 2

    def body(x_ref, wq_ref, wo_ref, wk_ref, wv_ref, out_ref,
             xg_ref, qg_ref, kg_ref, vg_ref,
             sbD_ref, rb1_ref, rb2_ref, mR_ref, mL_ref, fL_ref, fR_ref,
             snd, rcv):
        p = lax.axis_index("i")
        left = lax.rem(p + N_DEV - 1, N_DEV)
        right = lax.rem(p + 1, N_DEV)
        cm1, cp1, cp2 = left, right, lax.rem(p + 2, N_DEV)

        barrier = pltpu.get_barrier_semaphore()
        for nbr in (left, right):
            pl.semaphore_signal(barrier, inc=1, device_id=(nbr,),
                                device_id_type=pl.DeviceIdType.MESH)
        pl.semaphore_wait(barrier, 2)

        def rdma(src, dst, i, dev):
            return pltpu.make_async_remote_copy(
                src_ref=src, dst_ref=dst,
                send_sem=snd.at[i], recv_sem=rcv.at[i],
                device_id=(dev,), device_id_type=pl.DeviceIdType.MESH)

        bf16 = jnp.bfloat16

        own = pl.ds(p * s_per, s_per)
        xg_ref[own, :] = x_ref[0].astype(bf16)
        aR1 = rdma(xg_ref.at[own, :], xg_ref.at[own, :], AG_R1, right)
        aL1 = rdma(xg_ref.at[own, :], xg_ref.at[own, :], AG_L1, left)
        aR1.start()
        aL1.start()

        wqkv_bf = jnp.concatenate(
            [(wq_ref[...] * SCALE).astype(bf16),
             wk_ref[...].astype(bf16),
             wv_ref[...].astype(bf16)], axis=1)
        wo_bf = wo_ref[...].astype(bf16)

        def project(c):
            rows = pl.ds(c * s_per, s_per)
            qkv = jnp.dot(xg_ref[rows, :], wqkv_bf,
                          preferred_element_type=jnp.float32)
            qg_ref[rows, :] = qkv[:, :d].astype(bf16)
            kg_ref[rows, :] = qkv[:, d:2 * d].astype(bf16)
            vg_ref[rows, :] = qkv[:, 2 * d:].astype(bf16)

        project(p)
        aR1.wait_recv()
        aL1.wait_recv()
        h1m1 = pl.ds(cm1 * s_per, s_half)
        h2p1 = pl.ds(cp1 * s_per + s_half, s_half)
        aR2 = rdma(xg_ref.at[h1m1, :], xg_ref.at[h1m1, :], AG_R2, right)
        aL2 = rdma(xg_ref.at[h2p1, :], xg_ref.at[h2p1, :], AG_L2, left)
        aR2.start()
        aL2.start()
        project(cm1)
        project(cp1)
        aR2.wait_recv()
        aL2.wait_recv()
        project(cp2)

        ones_v = jnp.ones((seq,), dtype=bf16)

        def attn_part(c):
            rows = pl.ds(c * s_per, s_per)
            ohs = []
            for h in range(HEADS):
                cols = slice(h * DH, (h + 1) * DH)
                qh = qg_ref[rows, cols]
                kh = kg_ref[:, cols]
                vh = vg_ref[:, cols]
                s = lax.dot_general(qh, kh, (((1,), (1,)), ((), ())),
                                    preferred_element_type=jnp.float32)
                ef = jnp.exp(s)
                e = ef.astype(bf16)
                l = jnp.sum(ef, axis=-1)
                oh = (jnp.dot(e, vh, preferred_element_type=jnp.float32)
                      * (1.0 / l)[:, None])
                ohs.append(oh.astype(bf16))
            attn_c = jnp.concatenate(ohs, axis=1)
            return jnp.dot(attn_c, wo_bf, preferred_element_type=jnp.float32)

        sbD_ref[...] = attn_part(cp2).astype(bf16)
        bR1 = rdma(sbD_ref.at[pl.ds(0, s_half), :], rb1_ref, RS_R1, right)
        bL1 = rdma(sbD_ref.at[pl.ds(s_half, s_half), :], rb2_ref, RS_L1, left)
        bR1.start()
        bL1.start()

        partR = attn_part(cp1)
        bR1.wait_recv()
        mR_ref[pl.ds(0, s_half), :] = (
            partR[:s_half] + rb1_ref[...]).astype(bf16)
        mR_ref[pl.ds(s_half, s_half), :] = partR[s_half:].astype(bf16)
        bR2 = rdma(mR_ref, fL_ref, RS_R2, right)
        bR2.start()

        partL = attn_part(cm1)
        bL1.wait_recv()
        mL_ref[pl.ds(s_half, s_half), :] = (
            partL[s_half:] + rb2_ref[...]).astype(bf16)
        mL_ref[pl.ds(0, s_half), :] = partL[:s_half].astype(bf16)
        bL2 = rdma(mL_ref, fR_ref, RS_L2, left)
        bL2.start()

        part0 = attn_part(p)
        bR2.wait_recv()
        bL2.wait_recv()
        out_ref[0] = part0 + fL_ref[...] + fR_ref[...]

        for desc in (aR1, aL1, aR2, aL2, bR1, bL1, bR2, bL2):
            desc.wait_send()

    return pl.pallas_call(
        body,
        out_shape=jax.ShapeDtypeStruct((1, s_per, d), jnp.float32),
        in_specs=[pl.BlockSpec(memory_space=pltpu.VMEM)] * 5,
        out_specs=pl.BlockSpec(memory_space=pltpu.VMEM),
        scratch_shapes=[
            pltpu.VMEM((seq, d), jnp.bfloat16),
            pltpu.VMEM((seq, d), jnp.bfloat16),
            pltpu.VMEM((seq, d), jnp.bfloat16),
            pltpu.VMEM((seq, d), jnp.bfloat16),
            pltpu.VMEM((s_per, d), jnp.bfloat16),
            pltpu.VMEM((s_half, d), jnp.bfloat16),
            pltpu.VMEM((s_half, d), jnp.bfloat16),
            pltpu.VMEM((s_per, d), jnp.bfloat16),
            pltpu.VMEM((s_per, d), jnp.bfloat16),
            pltpu.VMEM((s_per, d), jnp.bfloat16),
            pltpu.VMEM((s_per, d), jnp.bfloat16),
            pltpu.SemaphoreType.DMA((8,)),
            pltpu.SemaphoreType.DMA((8,)),
        ],
        compiler_params=pltpu.CompilerParams(
            collective_id=0, vmem_limit_bytes=100 * 1024 * 1024),
    )(x, Wq, Wo, Wk, Wv)


# device time: 45339 ns/iter; 1.0751x vs baseline; 1.0019x over previous
import jax
import jax.numpy as jnp
from jax import lax
from jax.experimental import pallas as pl
from jax.experimental.pallas import tpu as pltpu

N_DEV = 4
HEADS = 8
DH = 128
SCALE = 0.08838834764831843

AG_R1, AG_L1, AG_R2, AG_L2, RS_R1, RS_L1, RS_R2, RS_L2 = range(8)


def kernel(x, Wq, Wo, Wk, Wv):
    _, s_per, d = x.shape
    seq = N_DEV * s_per
    s_half = s_per // 2

    def body(x_ref, wq_ref, wo_ref, wk_ref, wv_ref, out_ref,
             xg_ref, qg_ref, kg_ref, vg_ref,
             sbD_ref, rb1_ref, rb2_ref, mR_ref, mL_ref, fL_ref, fR_ref,
             snd, rcv):
        p = lax.axis_index("i")
        left = lax.rem(p + N_DEV - 1, N_DEV)
        right = lax.rem(p + 1, N_DEV)
        cm1, cp1, cp2 = left, right, lax.rem(p + 2, N_DEV)

        barrier = pltpu.get_barrier_semaphore()
        for nbr in (left, right):
            pl.semaphore_signal(barrier, inc=1, device_id=(nbr,),
                                device_id_type=pl.DeviceIdType.MESH)
        pl.semaphore_wait(barrier, 2)

        def rdma(src, dst, i, dev):
            return pltpu.make_async_remote_copy(
                src_ref=src, dst_ref=dst,
                send_sem=snd.at[i], recv_sem=rcv.at[i],
                device_id=(dev,), device_id_type=pl.DeviceIdType.MESH)

        bf16 = jnp.bfloat16

        own = pl.ds(p * s_per, s_per)
        xg_ref[own, :] = x_ref[0].astype(bf16)
        aR1 = rdma(xg_ref.at[own, :], xg_ref.at[own, :], AG_R1, right)
        aL1 = rdma(xg_ref.at[own, :], xg_ref.at[own, :], AG_L1, left)
        aR1.start()
        aL1.start()

        wqkv_bf = jnp.concatenate(
            [(wq_ref[...] * SCALE).astype(bf16),
             wk_ref[...].astype(bf16),
             wv_ref[...].astype(bf16)], axis=1)
        wo_bf = wo_ref[...].astype(bf16)

        def project(c):
            rows = pl.ds(c * s_per, s_per)
            qkv = jnp.dot(xg_ref[rows, :], wqkv_bf,
                          preferred_element_type=jnp.float32)
            qg_ref[rows, :] = qkv[:, :d].astype(bf16)
            kg_ref[rows, :] = qkv[:, d:2 * d].astype(bf16)
            vg_ref[rows, :] = qkv[:, 2 * d:].astype(bf16)

        project(p)
        aR1.wait_recv()
        aL1.wait_recv()
        h1m1 = pl.ds(cm1 * s_per, s_half)
        h2p1 = pl.ds(cp1 * s_per + s_half, s_half)
        aR2 = rdma(xg_ref.at[h1m1, :], xg_ref.at[h1m1, :], AG_R2, right)
        aL2 = rdma(xg_ref.at[h2p1, :], xg_ref.at[h2p1, :], AG_L2, left)
        aR2.start()
        aL2.start()
        project(cm1)
        project(cp1)
        aR2.wait_recv()
        aL2.wait_recv()
        project(cp2)

        def attn_part(c):
            rows = pl.ds(c * s_per, s_per)
            ohs = []
            for h in range(HEADS):
                cols = slice(h * DH, (h + 1) * DH)
                qh = qg_ref[rows, cols]
                kh = kg_ref[:, cols]
                vh = vg_ref[:, cols]
                s = lax.dot_general(qh, kh, (((1,), (1,)), ((), ())),
                                    preferred_element_type=jnp.float32)
                e = jnp.exp(s.astype(bf16))
                l = jnp.sum(e, axis=-1, dtype=jnp.float32)
                oh = (jnp.dot(e, vh, preferred_element_type=jnp.float32)
                      * (1.0 / l)[:, None])
                ohs.append(oh.astype(bf16))
            attn_c = jnp.concatenate(ohs, axis=1)
            return jnp.dot(attn_c, wo_bf, preferred_element_type=jnp.float32)

        sbD_ref[...] = attn_part(cp2).astype(bf16)
        bR1 = rdma(sbD_ref.at[pl.ds(0, s_half), :], rb1_ref, RS_R1, right)
        bL1 = rdma(sbD_ref.at[pl.ds(s_half, s_half), :], rb2_ref, RS_L1, left)
        bR1.start()
        bL1.start()

        partR = attn_part(cp1)
        bR1.wait_recv()
        mR_ref[pl.ds(0, s_half), :] = (
            partR[:s_half] + rb1_ref[...]).astype(bf16)
        mR_ref[pl.ds(s_half, s_half), :] = partR[s_half:].astype(bf16)
        bR2 = rdma(mR_ref, fL_ref, RS_R2, right)
        bR2.start()

        partL = attn_part(cm1)
        bL1.wait_recv()
        mL_ref[pl.ds(s_half, s_half), :] = (
            partL[s_half:] + rb2_ref[...]).astype(bf16)
        mL_ref[pl.ds(0, s_half), :] = partL[:s_half].astype(bf16)
        bL2 = rdma(mL_ref, fR_ref, RS_L2, left)
        bL2.start()

        part0 = attn_part(p)
        bR2.wait_recv()
        bL2.wait_recv()
        out_ref[0] = part0 + fL_ref[...] + fR_ref[...]

        for desc in (aR1, aL1, aR2, aL2, bR1, bL1, bR2, bL2):
            desc.wait_send()

    return pl.pallas_call(
        body,
        out_shape=jax.ShapeDtypeStruct((1, s_per, d), jnp.float32),
        in_specs=[pl.BlockSpec(memory_space=pltpu.VMEM)] * 5,
        out_specs=pl.BlockSpec(memory_space=pltpu.VMEM),
        scratch_shapes=[
            pltpu.VMEM((seq, d), jnp.bfloat16),
            pltpu.VMEM((seq, d), jnp.bfloat16),
            pltpu.VMEM((seq, d), jnp.bfloat16),
            pltpu.VMEM((seq, d), jnp.bfloat16),
            pltpu.VMEM((s_per, d), jnp.bfloat16),
            pltpu.VMEM((s_half, d), jnp.bfloat16),
            pltpu.VMEM((s_half, d), jnp.bfloat16),
            pltpu.VMEM((s_per, d), jnp.bfloat16),
            pltpu.VMEM((s_per, d), jnp.bfloat16),
            pltpu.VMEM((s_per, d), jnp.bfloat16),
            pltpu.VMEM((s_per, d), jnp.bfloat16),
            pltpu.SemaphoreType.DMA((8,)),
            pltpu.SemaphoreType.DMA((8,)),
        ],
        compiler_params=pltpu.CompilerParams(
            collective_id=0, vmem_limit_bytes=100 * 1024 * 1024),
    )(x, Wq, Wo, Wk, Wv)


# device time: 45333 ns/iter; 1.0753x vs baseline; 1.0001x over previous
import jax
import jax.numpy as jnp
from jax import lax
from jax.experimental import pallas as pl
from jax.experimental.pallas import tpu as pltpu

N_DEV = 4
HEADS = 8
DH = 128
SCALE = 0.08838834764831843

AG_R1, AG_L1, AG_R2, AG_L2, RS_R1, RS_L1, RS_R2, RS_L2 = range(8)


def kernel(x, Wq, Wo, Wk, Wv):
    _, s_per, d = x.shape
    seq = N_DEV * s_per
    s_half = s_per // 2

    def body(x_ref, wq_ref, wo_ref, wk_ref, wv_ref, out_ref,
             xg_ref, qg_ref, kg_ref, vg_ref,
             sbD_ref, rb1_ref, rb2_ref, mR_ref, mL_ref, fL_ref, fR_ref,
             snd, rcv):
        p = lax.axis_index("i")
        left = lax.rem(p + N_DEV - 1, N_DEV)
        right = lax.rem(p + 1, N_DEV)
        cm1, cp1, cp2 = left, right, lax.rem(p + 2, N_DEV)

        barrier = pltpu.get_barrier_semaphore()
        for nbr in (left, right):
            pl.semaphore_signal(barrier, inc=1, device_id=(nbr,),
                                device_id_type=pl.DeviceIdType.MESH)
        pl.semaphore_wait(barrier, 2)

        def rdma(src, dst, i, dev):
            return pltpu.make_async_remote_copy(
                src_ref=src, dst_ref=dst,
                send_sem=snd.at[i], recv_sem=rcv.at[i],
                device_id=(dev,), device_id_type=pl.DeviceIdType.MESH)

        bf16 = jnp.bfloat16

        own = pl.ds(p * s_per, s_per)
        xg_ref[own, :] = x_ref[0].astype(bf16)
        aR1 = rdma(xg_ref.at[own, :], xg_ref.at[own, :], AG_R1, right)
        aL1 = rdma(xg_ref.at[own, :], xg_ref.at[own, :], AG_L1, left)
        aR1.start()
        aL1.start()

        wqkv_bf = jnp.concatenate(
            [(wq_ref[...] * SCALE).astype(bf16),
             wk_ref[...].astype(bf16),
             wv_ref[...].astype(bf16)], axis=1)
        wo_bf = wo_ref[...].astype(bf16)

        def project(c):
            rows = pl.ds(c * s_per, s_per)
            qkv = jnp.dot(xg_ref[rows, :], wqkv_bf,
                          preferred_element_type=jnp.float32)
            qg_ref[rows, :] = qkv[:, :d].astype(bf16)
            kg_ref[rows, :] = qkv[:, d:2 * d].astype(bf16)
            vg_ref[rows, :] = qkv[:, 2 * d:].astype(bf16)

        project(p)
        aR1.wait_recv()
        aL1.wait_recv()
        h1m1 = pl.ds(cm1 * s_per, s_half)
        h2p1 = pl.ds(cp1 * s_per + s_half, s_half)
        aR2 = rdma(xg_ref.at[h1m1, :], xg_ref.at[h1m1, :], AG_R2, right)
        aL2 = rdma(xg_ref.at[h2p1, :], xg_ref.at[h2p1, :], AG_L2, left)
        aR2.start()
        aL2.start()
        project(cm1)
        project(cp1)
        aR2.wait_recv()
        aL2.wait_recv()
        project(cp2)

        def attn_part(c):
            rows = pl.ds(c * s_per, s_per)
            ohs = []
            for h in range(HEADS):
                cols = slice(h * DH, (h + 1) * DH)
                qh = qg_ref[rows, cols]
                kh = kg_ref[:, cols]
                vh = vg_ref[:, cols]
                s = lax.dot_general(qh, kh, (((1,), (1,)), ((), ())),
                                    preferred_element_type=jnp.float32)
                ef = jnp.exp(s)
                e = ef.astype(bf16)
                l = jnp.sum(ef, axis=-1)
                oh = (jnp.dot(e, vh, preferred_element_type=jnp.float32)
                      * (1.0 / l)[:, None])
                ohs.append(oh.astype(bf16))
            attn_c = jnp.concatenate(ohs, axis=1)
            return jnp.dot(attn_c, wo_bf, preferred_element_type=jnp.float32)

        sbD_ref[...] = attn_part(cp2).astype(bf16)
        bR1 = rdma(sbD_ref.at[pl.ds(0, s_half), :], rb1_ref, RS_R1, right)
        bL1 = rdma(sbD_ref.at[pl.ds(s_half, s_half), :], rb2_ref, RS_L1, left)
        bR1.start()
        bL1.start()

        partR = attn_part(cp1)
        bR1.wait_recv()
        mR_ref[pl.ds(0, s_half), :] = (
            partR[:s_half] + rb1_ref[...]).astype(bf16)
        mR_ref[pl.ds(s_half, s_half), :] = partR[s_half:].astype(bf16)
        bR2 = rdma(mR_ref, fL_ref, RS_R2, right)
        bR2.start()

        partL = attn_part(cm1)
        bL1.wait_recv()
        mL_ref[pl.ds(s_half, s_half), :] = (
            partL[s_half:] + rb2_ref[...]).astype(bf16)
        mL_ref[pl.ds(0, s_half), :] = partL[:s_half].astype(bf16)
        bL2 = rdma(mL_ref, fR_ref, RS_L2, left)
        bL2.start()

        part0 = attn_part(p)
        bR2.wait_recv()
        bL2.wait_recv()
        out_ref[0] = part0 + fL_ref[...] + fR_ref[...]

        for desc in (aR1, aL1, aR2, aL2, bR1, bL1, bR2, bL2):
            desc.wait_send()

    return pl.pallas_call(
        body,
        out_shape=jax.ShapeDtypeStruct((1, s_per, d), jnp.float32),
        in_specs=[pl.BlockSpec(memory_space=pltpu.VMEM)] * 5,
        out_specs=pl.BlockSpec(memory_space=pltpu.VMEM),
        scratch_shapes=[
            pltpu.VMEM((seq, d), jnp.bfloat16),
            pltpu.VMEM((seq, d), jnp.bfloat16),
            pltpu.VMEM((seq, d), jnp.bfloat16),
            pltpu.VMEM((seq, d), jnp.bfloat16),
            pltpu.VMEM((s_per, d), jnp.bfloat16),
            pltpu.VMEM((s_half, d), jnp.bfloat16),
            pltpu.VMEM((s_half, d), jnp.bfloat16),
            pltpu.VMEM((s_per, d), jnp.bfloat16),
            pltpu.VMEM((s_per, d), jnp.bfloat16),
            pltpu.VMEM((s_per, d), jnp.bfloat16),
            pltpu.VMEM((s_per, d), jnp.bfloat16),
            pltpu.SemaphoreType.DMA((8,)),
            pltpu.SemaphoreType.DMA((8,)),
        ],
        compiler_params=pltpu.CompilerParams(
            collective_id=0, vmem_limit_bytes=100 * 1024 * 1024),
    )(x, Wq, Wo, Wk, Wv)
